# baseline (speedup 1.0000x reference)
import functools
import jax
import jax.numpy as jnp
from jax import lax
from jax.experimental import pallas as pl
from jax.experimental.pallas import tpu as pltpu

N_NODES = 10000
N_EDGES = 320000
D_FEAT = 128
EPS = 1e-05

NB = 2048
EB = 2000
N_BLOCKS = 5
E_BLOCKS = N_EDGES // EB
N_PAD = NB * N_BLOCKS


def _body(dst_ref, src_ref, o_ref, s_acc, q_acc, d_acc):
    n = pl.program_id(0)
    e = pl.program_id(1)

    @pl.when(e == 0)
    def _():
        s_acc[...] = jnp.zeros_like(s_acc)
        q_acc[...] = jnp.zeros_like(q_acc)
        d_acc[...] = jnp.zeros_like(d_acc)

    ids = dst_ref[0, 0]
    rows = src_ref[...]
    node0 = n * NB
    node_ids = node0 + lax.broadcasted_iota(jnp.int32, (EB, NB), 1)
    onehot = (ids[:, None] == node_ids).astype(jnp.float32)
    dims = (((0,), (0,)), ((), ()))
    s_acc[...] += lax.dot_general(onehot, rows, dims,
                                  preferred_element_type=jnp.float32)
    q_acc[...] += lax.dot_general(onehot, rows * rows, dims,
                                  preferred_element_type=jnp.float32)
    d_acc[...] += jnp.sum(onehot, axis=0)[:, None]

    @pl.when(e == E_BLOCKS - 1)
    def _():
        deg = d_acc[...]
        safe = jnp.maximum(deg, 1.0)
        mean = s_acc[...] / safe
        msq = q_acc[...] / safe
        var = jnp.maximum(msq - mean * mean, 0.0)
        h = jnp.sqrt(var + EPS)
        o_ref[...] = jnp.where(deg > 0.0, h, 0.0)


@jax.jit
def _tc(dst, src):
    return pl.pallas_call(
        _body,
        out_shape=jax.ShapeDtypeStruct((N_PAD, D_FEAT), jnp.float32),
        grid=(N_BLOCKS, E_BLOCKS),
        in_specs=[
            pl.BlockSpec((1, 1, EB), lambda n, e: (e, 0, 0)),
            pl.BlockSpec((EB, D_FEAT), lambda n, e: (e, 0)),
        ],
        out_specs=pl.BlockSpec((NB, D_FEAT), lambda n, e: (n, 0)),
        scratch_shapes=[
            pltpu.VMEM((NB, D_FEAT), jnp.float32),
            pltpu.VMEM((NB, D_FEAT), jnp.float32),
            pltpu.VMEM((NB, 1), jnp.float32),
        ],
        compiler_params=pltpu.CompilerParams(
            dimension_semantics=("parallel", "arbitrary")),
    )(dst, src)


def kernel(src_emb, src_emb_in, edge_index):
    del src_emb_in
    dst_r = edge_index[1].reshape(E_BLOCKS, 1, EB)
    h = _tc(dst_r, src_emb)
    return h[:N_NODES]
